# baseline (speedup 1.0000x reference)
import functools

import jax
import jax.numpy as jnp
from jax import lax
from jax.experimental import pallas as pl
from jax.experimental.pallas import tpu as pltpu
from jax.experimental.pallas import tpu_sc as plsc

N = 10000
NP = 10240
E = 320000
F = 128
DP = 16
NC = 2
NS = 16
CH = 128
EP = 327680
NCH = EP // NS // CH
NB = 3
WH = 4
HALF = NP // 2
HPT = HALF // NS

_mesh = plsc.VectorSubcoreMesh(core_axis_name="c", subcore_axis_name="s")


@functools.partial(
    pl.kernel,
    out_type=jax.ShapeDtypeStruct((NP, F), jnp.float32),
    mesh=_mesh,
    scratch_types=[
        pltpu.VMEM((NCH, CH), jnp.int32),
        pltpu.VMEM((CH, F), jnp.float32),
        pltpu.VMEM_SHARED((HALF, F), jnp.float32),
        pltpu.SemaphoreType.DMA,
    ],
)
def _sc_degree(dst_hbm, ones_hbm, zero_hbm, degp_hbm, dv, ones_v, acc, ssem):
    c = lax.axis_index("c")
    s = lax.axis_index("s")

    pltpu.sync_copy(ones_hbm, ones_v)
    pltpu.sync_copy(zero_hbm, acc.at[pl.ds(s * HPT, HPT)])
    plsc.subcore_barrier()

    pltpu.sync_copy(dst_hbm.at[c, s], dv)

    def s_start(j):
        pltpu.async_copy(
            ones_v, acc.at[plsc.Indices(dv.at[j], ignored_value=-1)], ssem,
            add=True,
        )

    def s_wait(j):
        pltpu.make_async_copy(
            ones_v, acc.at[plsc.Indices(dv.at[j], ignored_value=-1)], ssem
        ).wait()

    for j in range(WH):
        s_start(j)

    def chunk(j, _):
        s_wait(j - WH)
        s_start(j)
        return _

    lax.fori_loop(WH, NCH, chunk, None)
    for b in range(WH):
        s_wait(NCH - WH + b)
    plsc.subcore_barrier()
    pltpu.sync_copy(
        acc.at[pl.ds(s * HPT, HPT)],
        degp_hbm.at[pl.ds(c * HALF + s * HPT, HPT)],
    )


@functools.partial(
    pl.kernel,
    out_type=jax.ShapeDtypeStruct((NP, F), jnp.float32),
    mesh=_mesh,
    scratch_types=[
        pltpu.VMEM((NCH, CH), jnp.int32),
        pltpu.VMEM((NCH, CH), jnp.int32),
        pltpu.VMEM((NB, CH, F), jnp.float32),
        pltpu.VMEM_SHARED((HALF, F), jnp.float32),
        pltpu.SemaphoreType.DMA((NB,)),
        pltpu.SemaphoreType.DMA((NB,)),
    ],
)
def _sc_agg_wide(p1_hbm, src_hbm, dst_hbm, agg_hbm, sv, dv, rows, acc, gs, ss):
    c = lax.axis_index("c")
    s = lax.axis_index("s")

    pltpu.sync_copy(
        p1_hbm.at[pl.ds(c * HALF + s * HPT, HPT)], acc.at[pl.ds(s * HPT, HPT)]
    )
    plsc.subcore_barrier()

    pltpu.sync_copy(src_hbm.at[c, s], sv)
    pltpu.sync_copy(dst_hbm.at[c, s], dv)

    def g_start(j, b):
        pltpu.async_copy(
            p1_hbm.at[plsc.Indices(sv.at[j], ignored_value=-1)], rows.at[b],
            gs.at[b],
        )

    def g_wait(j, b):
        pltpu.make_async_copy(
            p1_hbm.at[plsc.Indices(sv.at[j], ignored_value=-1)], rows.at[b],
            gs.at[b],
        ).wait()

    def s_start(j, b):
        pltpu.async_copy(
            rows.at[b], acc.at[plsc.Indices(dv.at[j], ignored_value=-1)],
            ss.at[b], add=True,
        )

    def s_wait(j, b):
        pltpu.make_async_copy(
            rows.at[b], acc.at[plsc.Indices(dv.at[j], ignored_value=-1)],
            ss.at[b],
        ).wait()

    def prologue(j, _):
        g_start(j, j)
        return _

    lax.fori_loop(0, NB, prologue, None)

    def main(j, _):
        b = lax.rem(j, NB)
        g_wait(j, b)
        s_start(j, b)
        s_wait(j, b)
        g_start(j + NB, b)
        return _

    lax.fori_loop(0, NCH - NB, main, None)

    def tail(j, _):
        b = lax.rem(j, NB)
        g_wait(j, b)
        s_start(j, b)
        s_wait(j, b)
        return _

    lax.fori_loop(NCH - NB, NCH, tail, None)
    plsc.subcore_barrier()
    pltpu.sync_copy(
        acc.at[pl.ds(s * HPT, HPT)],
        agg_hbm.at[pl.ds(c * HALF + s * HPT, HPT)],
    )


_BN = 1024


def _dis_block(degp_ref):
    return lax.rsqrt(degp_ref[:, 0:1] + 1.0)


def _tc1_body(x_ref, w1_ref, degp_ref, out_ref):
    dis = _dis_block(degp_ref)
    h = jnp.dot(x_ref[...], w1_ref[...], preferred_element_type=jnp.float32)
    out_ref[...] = h * dis


def _tc1(x, w1, degp):
    return pl.pallas_call(
        _tc1_body,
        grid=(NP // _BN,),
        in_specs=[
            pl.BlockSpec((_BN, F), lambda i: (i, 0)),
            pl.BlockSpec((F, F), lambda i: (0, 0)),
            pl.BlockSpec((_BN, F), lambda i: (i, 0)),
        ],
        out_specs=pl.BlockSpec((_BN, F), lambda i: (i, 0)),
        out_shape=jax.ShapeDtypeStruct((NP, F), jnp.float32),
    )(x, w1, degp)


def _tc2_body(agg_ref, degp_ref, b1_ref, w2_ref, out_ref):
    dis = _dis_block(degp_ref)
    h1 = jnp.maximum(dis * agg_ref[...] + b1_ref[0:1, :], 0.0)
    g = jnp.dot(h1, w2_ref[...], preferred_element_type=jnp.float32)
    out_ref[...] = dis * g


def _tc2(agg1, degp, b1, w2p):
    return pl.pallas_call(
        _tc2_body,
        grid=(NP // _BN,),
        in_specs=[
            pl.BlockSpec((_BN, F), lambda i: (i, 0)),
            pl.BlockSpec((_BN, F), lambda i: (i, 0)),
            pl.BlockSpec((1, F), lambda i: (0, 0)),
            pl.BlockSpec((F, F), lambda i: (0, 0)),
        ],
        out_specs=pl.BlockSpec((_BN, F), lambda i: (i, 0)),
        out_shape=jax.ShapeDtypeStruct((NP, F), jnp.float32),
    )(agg1, degp, b1, w2p)


def _tc3_body(agg_ref, degp_ref, b2_ref, out_ref):
    dis = _dis_block(degp_ref)
    out_ref[...] = dis * agg_ref[...] + b2_ref[0:1, :]


def _tc3(agg2, degp, b2p):
    return pl.pallas_call(
        _tc3_body,
        grid=(NP // _BN,),
        in_specs=[
            pl.BlockSpec((_BN, F), lambda i: (i, 0)),
            pl.BlockSpec((_BN, F), lambda i: (i, 0)),
            pl.BlockSpec((1, F), lambda i: (0, 0)),
        ],
        out_specs=pl.BlockSpec((_BN, F), lambda i: (i, 0)),
        out_shape=jax.ShapeDtypeStruct((NP, F), jnp.float32),
    )(agg2, degp, b2p)


def kernel(x, edge_index, W1, b1, W2, b2):
    src = edge_index[0].astype(jnp.int32)
    dst = edge_index[1].astype(jnp.int32)

    own0 = dst < HALF
    pad = jnp.full((EP - E,), -1, jnp.int32)

    def shard(idx):
        return jnp.concatenate([idx, pad]).reshape(NS, NCH, CH)

    src_sc = jnp.stack([
        shard(jnp.where(own0, src, -1)),
        shard(jnp.where(own0, -1, src)),
    ])
    dst_sc = jnp.stack([
        shard(jnp.where(own0, dst, -1)),
        shard(jnp.where(own0, -1, dst - HALF)),
    ])

    b1r = b1.reshape(1, F)
    w2p = jnp.pad(W2, ((0, 0), (0, F - W2.shape[1])))
    b2p = jnp.pad(b2, (0, F - b2.shape[0])).reshape(1, F)
    xp = jnp.pad(x, ((0, NP - N), (0, 0)))

    ones_c = jnp.ones((CH, F), jnp.float32)
    zero_c = jnp.zeros((HPT, F), jnp.float32)

    degp = _sc_degree(dst_sc, ones_c, zero_c)
    p1 = _tc1(xp, W1, degp)
    agg1 = _sc_agg_wide(p1, src_sc, dst_sc)
    p2 = _tc2(agg1, degp, b1r, w2p)
    agg2 = _sc_agg_wide(p2, src_sc, dst_sc)
    out = _tc3(agg2, degp, b2p)
    return out[:N, : W2.shape[1]]


# speedup vs baseline: 1.0131x; 1.0131x over previous
import functools

import jax
import jax.numpy as jnp
from jax import lax
from jax.experimental import pallas as pl
from jax.experimental.pallas import tpu as pltpu
from jax.experimental.pallas import tpu_sc as plsc

N = 10000
NP = 10240
E = 320000
F = 128
NC = 2
NS = 16
CH = 128
EP = 327680
NCH = EP // NS // CH
NB = 3
WH = 4
HALF = NP // 2
HPT = HALF // NS

_mesh = plsc.VectorSubcoreMesh(
    core_axis_name="c", subcore_axis_name="s", num_cores=NC, num_subcores=NS
)


@functools.partial(
    pl.kernel,
    out_type=jax.ShapeDtypeStruct((NP, F), jnp.float32),
    mesh=_mesh,
    scratch_types=[
        pltpu.VMEM((NCH, CH), jnp.int32),
        pltpu.VMEM((CH, F), jnp.float32),
        pltpu.VMEM_SHARED((HALF, F), jnp.float32),
        pltpu.SemaphoreType.DMA,
    ],
)
def _sc_degree(dst_hbm, ones_hbm, zero_hbm, degp_hbm, dv, ones_v, acc, ssem):
    c = lax.axis_index("c")
    s = lax.axis_index("s")

    pltpu.sync_copy(ones_hbm, ones_v)
    pltpu.sync_copy(zero_hbm, acc.at[pl.ds(s * HPT, HPT)])
    plsc.subcore_barrier()

    pltpu.sync_copy(dst_hbm.at[c, s], dv)

    def s_start(j):
        pltpu.async_copy(
            ones_v, acc.at[plsc.Indices(dv.at[j], ignored_value=-1)], ssem,
            add=True,
        )

    def s_wait(j):
        pltpu.make_async_copy(
            ones_v, acc.at[plsc.Indices(dv.at[j], ignored_value=-1)], ssem
        ).wait()

    for j in range(WH):
        s_start(j)

    def chunk(j, _):
        s_wait(j - WH)
        s_start(j)
        return _

    lax.fori_loop(WH, NCH, chunk, None)
    for b in range(WH):
        s_wait(NCH - WH + b)
    plsc.subcore_barrier()
    pltpu.sync_copy(
        acc.at[pl.ds(s * HPT, HPT)],
        degp_hbm.at[pl.ds(c * HALF + s * HPT, HPT)],
    )


@functools.partial(
    pl.kernel,
    out_type=jax.ShapeDtypeStruct((NP, F), jnp.float32),
    mesh=_mesh,
    scratch_types=[
        pltpu.VMEM((NCH, CH), jnp.int32),
        pltpu.VMEM((NCH, CH), jnp.int32),
        pltpu.VMEM((NB, CH, F), jnp.float32),
        pltpu.VMEM_SHARED((HALF, F), jnp.float32),
        pltpu.SemaphoreType.DMA((NB,)),
        pltpu.SemaphoreType.DMA((NB,)),
    ],
)
def _sc_agg_wide(p1_hbm, src_hbm, dst_hbm, agg_hbm, sv, dv, rows, acc, gs, ss):
    c = lax.axis_index("c")
    s = lax.axis_index("s")

    pltpu.sync_copy(
        p1_hbm.at[pl.ds(c * HALF + s * HPT, HPT)], acc.at[pl.ds(s * HPT, HPT)]
    )
    plsc.subcore_barrier()

    pltpu.sync_copy(src_hbm.at[c, s], sv)
    pltpu.sync_copy(dst_hbm.at[c, s], dv)

    def g_start(j, b):
        pltpu.async_copy(
            p1_hbm.at[plsc.Indices(sv.at[j], ignored_value=-1)], rows.at[b],
            gs.at[b],
        )

    def g_wait(j, b):
        pltpu.make_async_copy(
            p1_hbm.at[plsc.Indices(sv.at[j], ignored_value=-1)], rows.at[b],
            gs.at[b],
        ).wait()

    def s_start(j, b):
        pltpu.async_copy(
            rows.at[b], acc.at[plsc.Indices(dv.at[j], ignored_value=-1)],
            ss.at[b], add=True,
        )

    def s_wait(j, b):
        pltpu.make_async_copy(
            rows.at[b], acc.at[plsc.Indices(dv.at[j], ignored_value=-1)],
            ss.at[b],
        ).wait()

    def prologue(j, _):
        g_start(j, j)
        return _

    lax.fori_loop(0, NB, prologue, None)

    def main(j, _):
        b = lax.rem(j, NB)
        g_wait(j, b)
        s_start(j, b)
        s_wait(j, b)
        g_start(j + NB, b)
        return _

    lax.fori_loop(0, NCH - NB, main, None)

    def tail(j, _):
        b = lax.rem(j, NB)
        g_wait(j, b)
        s_start(j, b)
        s_wait(j, b)
        return _

    lax.fori_loop(NCH - NB, NCH, tail, None)
    plsc.subcore_barrier()
    pltpu.sync_copy(
        acc.at[pl.ds(s * HPT, HPT)],
        agg_hbm.at[pl.ds(c * HALF + s * HPT, HPT)],
    )


_BN = 1024


def _dis_block(degp_ref):
    return lax.rsqrt(degp_ref[:, 0:1] + 1.0)


def _tc1_body(x_ref, w1_ref, degp_ref, out_ref):
    dis = _dis_block(degp_ref)
    h = jnp.dot(x_ref[...], w1_ref[...], preferred_element_type=jnp.float32)
    out_ref[...] = h * dis


def _tc1(x, w1, degp):
    return pl.pallas_call(
        _tc1_body,
        grid=(NP // _BN,),
        in_specs=[
            pl.BlockSpec((_BN, F), lambda i: (i, 0)),
            pl.BlockSpec((F, F), lambda i: (0, 0)),
            pl.BlockSpec((_BN, F), lambda i: (i, 0)),
        ],
        out_specs=pl.BlockSpec((_BN, F), lambda i: (i, 0)),
        out_shape=jax.ShapeDtypeStruct((NP, F), jnp.float32),
    )(x, w1, degp)


def _tc2_body(agg_ref, degp_ref, b1_ref, w2_ref, out_ref):
    dis = _dis_block(degp_ref)
    h1 = jnp.maximum(dis * agg_ref[...] + b1_ref[0:1, :], 0.0)
    g = jnp.dot(h1, w2_ref[...], preferred_element_type=jnp.float32)
    out_ref[...] = dis * g


def _tc2(agg1, degp, b1, w2p):
    return pl.pallas_call(
        _tc2_body,
        grid=(NP // _BN,),
        in_specs=[
            pl.BlockSpec((_BN, F), lambda i: (i, 0)),
            pl.BlockSpec((_BN, F), lambda i: (i, 0)),
            pl.BlockSpec((1, F), lambda i: (0, 0)),
            pl.BlockSpec((F, F), lambda i: (0, 0)),
        ],
        out_specs=pl.BlockSpec((_BN, F), lambda i: (i, 0)),
        out_shape=jax.ShapeDtypeStruct((NP, F), jnp.float32),
    )(agg1, degp, b1, w2p)


def _tc3_body(agg_ref, degp_ref, b2_ref, out_ref):
    dis = _dis_block(degp_ref)
    out_ref[...] = dis * agg_ref[...] + b2_ref[0:1, :]


def _tc3(agg2, degp, b2p):
    return pl.pallas_call(
        _tc3_body,
        grid=(NP // _BN,),
        in_specs=[
            pl.BlockSpec((_BN, F), lambda i: (i, 0)),
            pl.BlockSpec((_BN, F), lambda i: (i, 0)),
            pl.BlockSpec((1, F), lambda i: (0, 0)),
        ],
        out_specs=pl.BlockSpec((_BN, F), lambda i: (i, 0)),
        out_shape=jax.ShapeDtypeStruct((NP, F), jnp.float32),
    )(agg2, degp, b2p)


def kernel(x, edge_index, W1, b1, W2, b2):
    src = edge_index[0].astype(jnp.int32)
    dst = edge_index[1].astype(jnp.int32)

    own0 = dst < HALF
    pad = jnp.full((EP - E,), -1, jnp.int32)

    def shard(idx):
        return jnp.concatenate([idx, pad]).reshape(NS, NCH, CH)

    src_sc = jnp.stack([
        shard(jnp.where(own0, src, -1)),
        shard(jnp.where(own0, -1, src)),
    ])
    dst_sc = jnp.stack([
        shard(jnp.where(own0, dst, -1)),
        shard(jnp.where(own0, -1, dst - HALF)),
    ])

    b1r = b1.reshape(1, F)
    w2p = jnp.pad(W2, ((0, 0), (0, F - W2.shape[1])))
    b2p = jnp.pad(b2, (0, F - b2.shape[0])).reshape(1, F)
    xp = jnp.pad(x, ((0, NP - N), (0, 0)))

    ones_c = jnp.ones((CH, F), jnp.float32)
    zero_c = jnp.zeros((HPT, F), jnp.float32)

    degp = _sc_degree(dst_sc, ones_c, zero_c)
    p1 = _tc1(xp, W1, degp)
    agg1 = _sc_agg_wide(p1, src_sc, dst_sc)
    p2 = _tc2(agg1, degp, b1r, w2p)
    agg2 = _sc_agg_wide(p2, src_sc, dst_sc)
    out = _tc3(agg2, degp, b2p)
    return out[:N, : W2.shape[1]]
